# baseline (speedup 1.0000x reference)
import functools

import jax
import jax.numpy as jnp
from jax import lax
from jax.experimental import pallas as pl
from jax.experimental.pallas import tpu as pltpu
from jax.experimental.pallas import tpu_sc as plsc

N = 10000
M = 5000
E = 320000
D = 128
DH = D // 2

NC = 2
NS = 16
CHUNK = 128
EPT = E // NS
EPT_PAD = ((EPT + CHUNK - 1) // CHUNK) * CHUNK
NCHUNK = EPT_PAD // CHUNK
E_PAD = NS * EPT_PAD

MP = 5120
NP = 10240


NBUF = 2


def _sc_segment_sum(op_rows, table, src_lo, src_hi, dst_idx):
    zrows = op_rows // NS
    zchunks = zrows // 64

    mesh = plsc.VectorSubcoreMesh(core_axis_name="c", subcore_axis_name="s")

    @functools.partial(
        pl.kernel,
        out_type=jax.ShapeDtypeStruct((NC, op_rows, DH), jnp.float32),
        mesh=mesh,
        compiler_params=pltpu.CompilerParams(use_tc_tiling_on_sc=False),
        scratch_types=[
            pltpu.VMEM((NCHUNK, CHUNK), jnp.int32),
            pltpu.VMEM((NCHUNK, CHUNK), jnp.int32),
            [pltpu.VMEM((CHUNK, DH), jnp.float32)] * NBUF,
            pltpu.VMEM((64, DH), jnp.float32),
            pltpu.VMEM_SHARED((op_rows, DH), jnp.float32),
            [pltpu.SemaphoreType.DMA] * NBUF,
        ],
    )
    def k(table_h, srclo_h, srchi_h, dst_h, out_h, srcv, dstv, rows,
          zv, acc, gsem):
        c = lax.axis_index("c")
        s = lax.axis_index("s")

        def zrow(r, _):
            def zcol(q, _):
                zv[r, pl.ds(q * 16, 16)] = jnp.zeros((16,), jnp.float32)
                return 0
            return lax.fori_loop(0, DH // 16, zcol, 0)
        lax.fori_loop(0, 64, zrow, 0)

        zbase = s * zrows

        def zcp(j, _):
            pltpu.sync_copy(zv, acc.at[pl.ds(zbase + j * 64, 64)])
            return 0
        lax.fori_loop(0, zchunks, zcp, 0)

        @pl.when(c == 0)
        def _():
            pltpu.sync_copy(srclo_h.at[s], srcv)

        @pl.when(c == 1)
        def _():
            pltpu.sync_copy(srchi_h.at[s], srcv)

        pltpu.sync_copy(dst_h.at[s], dstv)
        plsc.subcore_barrier()

        def gath(j, b):
            return pltpu.make_async_copy(table_h.at[srcv.at[j]], rows[b],
                                         gsem[b])

        gath(0, 0).start()

        def step(j, _):
            even = lax.rem(j, 2) == 0

            @pl.when(jnp.logical_and(j + 1 < NCHUNK, even))
            def _():
                gath(j + 1, 1).start()

            @pl.when(jnp.logical_and(j + 1 < NCHUNK, jnp.logical_not(even)))
            def _():
                gath(j + 1, 0).start()

            @pl.when(even)
            def _():
                gath(j, 0).wait()
                pltpu.sync_copy(rows[0], acc.at[dstv.at[j]], add=True)

            @pl.when(jnp.logical_not(even))
            def _():
                gath(j, 1).wait()
                pltpu.sync_copy(rows[1], acc.at[dstv.at[j]], add=True)

            return 0

        lax.fori_loop(0, NCHUNK, step, 0)
        plsc.subcore_barrier()

        def ocp(j, _):
            pltpu.sync_copy(acc.at[pl.ds(zbase + j * 64, 64)],
                            out_h.at[c, pl.ds(zbase + j * 64, 64)])
            return 0
        lax.fori_loop(0, zchunks, ocp, 0)

    return k(table.reshape(2 * table.shape[0], DH), src_lo, src_hi, dst_idx)


def _tc_prescale(x, w_col):
    rows = x.shape[0]
    blk = 1000
    grid = rows // blk

    def body(x_ref, w_ref, o_ref):
        o_ref[...] = x_ref[...] * w_ref[...]

    return pl.pallas_call(
        body,
        grid=(grid,),
        in_specs=[
            pl.BlockSpec((blk, D), lambda i: (i, 0)),
            pl.BlockSpec((blk, 1), lambda i: (i, 0)),
        ],
        out_specs=pl.BlockSpec((blk, D), lambda i: (i, 0)),
        out_shape=jax.ShapeDtypeStruct((rows, D), jnp.float32),
    )(x, w_col)


def _tc_finish(h0, h1, rsum, rw, W, b, want_scaled):
    rows = h0.shape[0]
    blk = 1000
    grid = rows // blk

    def body(h0_ref, h1_ref, rs_ref, rw_ref, w_ref, b_ref, o1_ref, o2_ref):
        acc = jnp.concatenate([h0_ref[...], h1_ref[...]], axis=1)
        x = acc / rs_ref[...]
        y = jnp.dot(x, w_ref[...], preferred_element_type=jnp.float32)
        y = jnp.maximum(y + b_ref[...], 0.0)
        o1_ref[...] = y
        o2_ref[...] = y * rw_ref[...]

    def body_single(h0_ref, h1_ref, rs_ref, w_ref, b_ref, o1_ref):
        acc = jnp.concatenate([h0_ref[...], h1_ref[...]], axis=1)
        x = acc / rs_ref[...]
        y = jnp.dot(x, w_ref[...], preferred_element_type=jnp.float32)
        y = jnp.maximum(y + b_ref[...], 0.0)
        o1_ref[...] = y

    half_spec = pl.BlockSpec((blk, DH), lambda i: (i, 0))
    row_spec = pl.BlockSpec((blk, D), lambda i: (i, 0))
    col_spec = pl.BlockSpec((blk, 1), lambda i: (i, 0))
    w_spec = pl.BlockSpec((D, D), lambda i: (0, 0))
    b_spec = pl.BlockSpec((1, D), lambda i: (0, 0))

    if want_scaled:
        return pl.pallas_call(
            body,
            grid=(grid,),
            in_specs=[half_spec, half_spec, col_spec, col_spec, w_spec, b_spec],
            out_specs=[row_spec, row_spec],
            out_shape=[jax.ShapeDtypeStruct((rows, D), jnp.float32),
                       jax.ShapeDtypeStruct((rows, D), jnp.float32)],
        )(h0, h1, rsum, rw, W, b)
    return pl.pallas_call(
        body_single,
        grid=(grid,),
        in_specs=[half_spec, half_spec, col_spec, w_spec, b_spec],
        out_specs=row_spec,
        out_shape=jax.ShapeDtypeStruct((rows, D), jnp.float32),
    )(h0, h1, rsum, W, b)


def _pad_idx(src, dst, junk_row):
    pad = E_PAD - E
    src2 = jnp.concatenate([src * 2, jnp.zeros((pad,), jnp.int32)])
    dst_p = jnp.concatenate([dst, jnp.full((pad,), junk_row, jnp.int32)])
    return (src2.reshape(NS, NCHUNK, CHUNK),
            (src2 + 1).reshape(NS, NCHUNK, CHUNK),
            dst_p.reshape(NS, NCHUNK, CHUNK))


def kernel(vfeat, efeat, v_reg_weight, v_reg_sum, e_reg_weight, e_reg_sum,
           g1_src, g1_dst, g2_src, g2_dst, W_ve, b_ve, W_ev, b_ev):
    g1lo, g1hi, g1d = _pad_idx(g1_src, g1_dst, MP - 1)
    g2lo, g2hi, g2d = _pad_idx(g2_src, g2_dst, NP - 1)
    b_ve2 = b_ve.reshape(1, D)
    b_ev2 = b_ev.reshape(1, D)

    scaled_vfeat = _tc_prescale(vfeat, v_reg_weight)
    p1 = _sc_segment_sum(MP, scaled_vfeat, g1lo, g1hi, g1d)
    efeat_new, scaled_efeat = _tc_finish(
        p1[0, :M], p1[1, :M], e_reg_sum, e_reg_weight, W_ve, b_ve2, True)

    p2 = _sc_segment_sum(NP, scaled_efeat, g2lo, g2hi, g2d)
    vfeat_new = _tc_finish(
        p2[0, :N], p2[1, :N], v_reg_sum, None, W_ev, b_ev2, False)

    return (vfeat_new, efeat_new)


# speedup vs baseline: 1.0096x; 1.0096x over previous
import functools

import jax
import jax.numpy as jnp
from jax import lax
from jax.experimental import pallas as pl
from jax.experimental.pallas import tpu as pltpu
from jax.experimental.pallas import tpu_sc as plsc

N = 10000
M = 5000
E = 320000
D = 128
DH = D // 2

NC = 2
NS = 16
CHUNK = 128
EPT = E // NS
EPT_PAD = ((EPT + CHUNK - 1) // CHUNK) * CHUNK
NCHUNK = EPT_PAD // CHUNK
E_PAD = NS * EPT_PAD

MP = 5120
NP = 10240


NBUF = 2


def _sc_segment_sum(op_rows, table, src2_idx, dst_idx):
    zrows = op_rows // NS
    zchunks = zrows // 64

    mesh = plsc.VectorSubcoreMesh(core_axis_name="c", subcore_axis_name="s")

    @functools.partial(
        pl.kernel,
        out_type=jax.ShapeDtypeStruct((NC, op_rows, DH), jnp.float32),
        mesh=mesh,
        compiler_params=pltpu.CompilerParams(use_tc_tiling_on_sc=False),
        scratch_types=[
            pltpu.VMEM((NCHUNK, CHUNK), jnp.int32),
            pltpu.VMEM((NCHUNK, CHUNK), jnp.int32),
            [pltpu.VMEM((CHUNK, DH), jnp.float32)] * NBUF,
            pltpu.VMEM((64, DH), jnp.float32),
            pltpu.VMEM_SHARED((op_rows, DH), jnp.float32),
            [pltpu.SemaphoreType.DMA] * NBUF,
        ],
    )
    def k(table_h, src_h, dst_h, out_h, srcv, dstv, rows, zv, acc, gsem):
        c = lax.axis_index("c")
        s = lax.axis_index("s")

        def zrow(r, _):
            def zcol(q, _):
                zv[r, pl.ds(q * 16, 16)] = jnp.zeros((16,), jnp.float32)
                return 0
            return lax.fori_loop(0, DH // 16, zcol, 0)
        lax.fori_loop(0, 64, zrow, 0)

        zbase = s * zrows

        def zcp(j, _):
            pltpu.sync_copy(zv, acc.at[pl.ds(zbase + j * 64, 64)])
            return 0
        lax.fori_loop(0, zchunks, zcp, 0)

        pltpu.sync_copy(src_h.at[s], srcv)
        pltpu.sync_copy(dst_h.at[s], dstv)

        def fixrow(j, _):
            def fixcol(q, _):
                sl = pl.ds(q * 16, 16)
                srcv[j, sl] = srcv[j, sl] + c
                return 0
            return lax.fori_loop(0, CHUNK // 16, fixcol, 0)
        lax.fori_loop(0, NCHUNK, fixrow, 0)
        plsc.subcore_barrier()

        def gath(j, b):
            return pltpu.make_async_copy(table_h.at[srcv.at[j]], rows[b],
                                         gsem[b])

        gath(0, 0).start()

        def step(j, _):
            even = lax.rem(j, 2) == 0

            @pl.when(jnp.logical_and(j + 1 < NCHUNK, even))
            def _():
                gath(j + 1, 1).start()

            @pl.when(jnp.logical_and(j + 1 < NCHUNK, jnp.logical_not(even)))
            def _():
                gath(j + 1, 0).start()

            @pl.when(even)
            def _():
                gath(j, 0).wait()
                pltpu.sync_copy(rows[0], acc.at[dstv.at[j]], add=True)

            @pl.when(jnp.logical_not(even))
            def _():
                gath(j, 1).wait()
                pltpu.sync_copy(rows[1], acc.at[dstv.at[j]], add=True)

            return 0

        lax.fori_loop(0, NCHUNK, step, 0)
        plsc.subcore_barrier()

        def ocp(j, _):
            pltpu.sync_copy(acc.at[pl.ds(zbase + j * 64, 64)],
                            out_h.at[c, pl.ds(zbase + j * 64, 64)])
            return 0
        lax.fori_loop(0, zchunks, ocp, 0)

    return k(table.reshape(2 * table.shape[0], DH), src2_idx, dst_idx)


def _tc_prescale(x, w_col):
    rows = x.shape[0]
    blk = 1000
    grid = rows // blk

    def body(x_ref, w_ref, o_ref):
        o_ref[...] = x_ref[...] * w_ref[...]

    return pl.pallas_call(
        body,
        grid=(grid,),
        in_specs=[
            pl.BlockSpec((blk, D), lambda i: (i, 0)),
            pl.BlockSpec((blk, 1), lambda i: (i, 0)),
        ],
        out_specs=pl.BlockSpec((blk, D), lambda i: (i, 0)),
        out_shape=jax.ShapeDtypeStruct((rows, D), jnp.float32),
    )(x, w_col)


def _tc_finish(h0, h1, rsum, rw, W, b, want_scaled):
    rows = h0.shape[0]
    blk = 1000
    grid = rows // blk

    def body(h0_ref, h1_ref, rs_ref, rw_ref, w_ref, b_ref, o1_ref, o2_ref):
        acc = jnp.concatenate([h0_ref[...], h1_ref[...]], axis=1)
        x = acc / rs_ref[...]
        y = jnp.dot(x, w_ref[...], preferred_element_type=jnp.float32)
        y = jnp.maximum(y + b_ref[...], 0.0)
        o1_ref[...] = y
        o2_ref[...] = y * rw_ref[...]

    def body_single(h0_ref, h1_ref, rs_ref, w_ref, b_ref, o1_ref):
        acc = jnp.concatenate([h0_ref[...], h1_ref[...]], axis=1)
        x = acc / rs_ref[...]
        y = jnp.dot(x, w_ref[...], preferred_element_type=jnp.float32)
        y = jnp.maximum(y + b_ref[...], 0.0)
        o1_ref[...] = y

    half_spec = pl.BlockSpec((blk, DH), lambda i: (i, 0))
    row_spec = pl.BlockSpec((blk, D), lambda i: (i, 0))
    col_spec = pl.BlockSpec((blk, 1), lambda i: (i, 0))
    w_spec = pl.BlockSpec((D, D), lambda i: (0, 0))
    b_spec = pl.BlockSpec((1, D), lambda i: (0, 0))

    if want_scaled:
        return pl.pallas_call(
            body,
            grid=(grid,),
            in_specs=[half_spec, half_spec, col_spec, col_spec, w_spec, b_spec],
            out_specs=[row_spec, row_spec],
            out_shape=[jax.ShapeDtypeStruct((rows, D), jnp.float32),
                       jax.ShapeDtypeStruct((rows, D), jnp.float32)],
        )(h0, h1, rsum, rw, W, b)
    return pl.pallas_call(
        body_single,
        grid=(grid,),
        in_specs=[half_spec, half_spec, col_spec, w_spec, b_spec],
        out_specs=row_spec,
        out_shape=jax.ShapeDtypeStruct((rows, D), jnp.float32),
    )(h0, h1, rsum, W, b)


def _pad_idx(src, dst, junk_row):
    pad = E_PAD - E
    src2 = jnp.concatenate([src * 2, jnp.zeros((pad,), jnp.int32)])
    dst_p = jnp.concatenate([dst, jnp.full((pad,), junk_row, jnp.int32)])
    return (src2.reshape(NS, NCHUNK, CHUNK), dst_p.reshape(NS, NCHUNK, CHUNK))


def kernel(vfeat, efeat, v_reg_weight, v_reg_sum, e_reg_weight, e_reg_sum,
           g1_src, g1_dst, g2_src, g2_dst, W_ve, b_ve, W_ev, b_ev):
    g1s, g1d = _pad_idx(g1_src, g1_dst, MP - 1)
    g2s, g2d = _pad_idx(g2_src, g2_dst, NP - 1)
    b_ve2 = b_ve.reshape(1, D)
    b_ev2 = b_ev.reshape(1, D)

    scaled_vfeat = _tc_prescale(vfeat, v_reg_weight)
    p1 = _sc_segment_sum(MP, scaled_vfeat, g1s, g1d)
    efeat_new, scaled_efeat = _tc_finish(
        p1[0, :M], p1[1, :M], e_reg_sum, e_reg_weight, W_ve, b_ve2, True)

    p2 = _sc_segment_sum(NP, scaled_efeat, g2s, g2d)
    vfeat_new = _tc_finish(
        p2[0, :N], p2[1, :N], v_reg_sum, None, W_ev, b_ev2, False)

    return (vfeat_new, efeat_new)
